# baseline (speedup 1.0000x reference)
import functools

import jax
import jax.numpy as jnp
from jax import lax
from jax.experimental import pallas as pl
from jax.experimental.pallas import tpu as pltpu
from jax.experimental.pallas import tpu_sc as plsc

N = 10000
NPAD = 10240
D = 256
DH = 128
E = 160000
EPAD = 163840
NCHUNK = EPAD // 128
NCHUNK2 = EPAD // 64
KA2 = NCHUNK2 // 16
KD = NCHUNK // 32
ROWS_PER_TILE = NPAD // 16

_MESH = plsc.VectorSubcoreMesh(
    core_axis_name="c", subcore_axis_name="s", num_cores=2, num_subcores=16)


def _sc_degree(dst2d):
    @functools.partial(
        pl.kernel,
        out_type=(jax.ShapeDtypeStruct((NPAD, 128), jnp.float32),
                  jax.ShapeDtypeStruct((NPAD, 128), jnp.float32)),
        mesh=_MESH,
        scratch_types=[
            pltpu.VMEM((KD, 128), jnp.int32),
            pltpu.VMEM((128, 128), jnp.float32),
            pltpu.VMEM((128, 128), jnp.float32),
            pltpu.VMEM_SHARED((NPAD, 128), jnp.float32),
        ],
    )
    def k(dst_hbm, dega_hbm, degb_hbm, dstv, ones_v, buf, acc):
        c = lax.axis_index("c")
        s = lax.axis_index("s")
        zero16 = jnp.zeros((16,), jnp.float32)
        one16 = jnp.ones((16,), jnp.float32)

        def zrow(r, carry):
            for g in range(8):
                buf[r, pl.ds(g * 16, 16)] = zero16
            return carry
        lax.fori_loop(0, 128, zrow, 0)

        def orow(r, carry):
            for g in range(8):
                ones_v[r, pl.ds(g * 16, 16)] = one16
            return carry
        lax.fori_loop(0, 128, orow, 0)

        for kb in range(ROWS_PER_TILE // 128):
            pltpu.sync_copy(buf, acc.at[pl.ds(s * ROWS_PER_TILE + kb * 128, 128)])
        plsc.subcore_barrier()

        pltpu.sync_copy(dst_hbm.at[pl.ds((c * 16 + s) * KD, KD)], dstv)

        def body(j, carry):
            pltpu.sync_copy(ones_v, acc.at[dstv.at[j]], add=True)
            return carry
        lax.fori_loop(0, KD, body, 0)

        plsc.subcore_barrier()

        def wout(out_hbm):
            for kb in range(ROWS_PER_TILE // 128):
                base = s * ROWS_PER_TILE + kb * 128
                pltpu.sync_copy(acc.at[pl.ds(base, 128)], buf)
                pltpu.sync_copy(buf, out_hbm.at[pl.ds(base, 128)])

        @pl.when(c == 0)
        def _():
            wout(dega_hbm)

        @pl.when(c == 1)
        def _():
            wout(degb_hbm)

    return k(dst2d)


def _sc_agg(zlo, zhi, src2d, dst2d):
    @functools.partial(
        pl.kernel,
        out_type=(jax.ShapeDtypeStruct((NPAD, DH), jnp.float32),
                  jax.ShapeDtypeStruct((NPAD, DH), jnp.float32)),
        mesh=_MESH,
        scratch_types=[
            pltpu.VMEM((KA2 // 4, 64), jnp.int32),
            pltpu.VMEM((KA2 // 4, 64), jnp.int32),
            pltpu.VMEM((64, DH), jnp.float32),
            pltpu.VMEM((64, DH), jnp.float32),
            pltpu.VMEM((64, DH), jnp.float32),
            pltpu.VMEM((64, DH), jnp.float32),
            pltpu.VMEM_SHARED((NPAD, DH), jnp.float32),
            pltpu.SemaphoreType.DMA,
            pltpu.SemaphoreType.DMA,
            pltpu.SemaphoreType.DMA,
            pltpu.SemaphoreType.DMA,
            pltpu.SemaphoreType.DMA,
            pltpu.SemaphoreType.DMA,
            pltpu.SemaphoreType.DMA,
            pltpu.SemaphoreType.DMA,
        ],
    )
    def k(zlo_hbm, zhi_hbm, src_hbm, dst_hbm, outlo, outhi,
          srcv, dstv, b0, b1, b2, b3, acc,
          sg0, sg1, sg2, sg3, ss0, ss1, ss2, ss3):
        c = lax.axis_index("c")
        s = lax.axis_index("s")
        zero16 = jnp.zeros((16,), jnp.float32)

        def zrow(r, carry):
            for g in range(DH // 16):
                b0[r, pl.ds(g * 16, 16)] = zero16
                b1[r, pl.ds(g * 16, 16)] = zero16
            return carry
        lax.fori_loop(0, 64, zrow, 0)
        for kb in range(ROWS_PER_TILE // 128):
            base = s * ROWS_PER_TILE + kb * 128
            pltpu.sync_copy(b0, acc.at[pl.ds(base, 64)])
            pltpu.sync_copy(b1, acc.at[pl.ds(base + 64, 64)])
        plsc.subcore_barrier()

        def run(z_hbm):
            kh = KA2 // 4
            bufs = (b0, b1, b2, b3)
            sgs = (sg0, sg1, sg2, sg3)
            sss = (ss0, ss1, ss2, ss3)
            for half in range(4):
                pltpu.sync_copy(src_hbm.at[pl.ds(s * KA2 + half * kh, kh)], srcv)
                pltpu.sync_copy(dst_hbm.at[pl.ds(s * KA2 + half * kh, kh)], dstv)
                nit = kh // 4

                def body(jj, carry):
                    j0 = jj * 4
                    gds = [pltpu.async_copy(z_hbm.at[srcv.at[j0 + q]],
                                            bufs[q], sgs[q])
                           for q in range(4)]
                    sds = []
                    for q in range(4):
                        gds[q].wait()
                        sds.append(pltpu.async_copy(
                            bufs[q], acc.at[dstv.at[j0 + q]], sss[q], add=True))
                    for q in range(4):
                        sds[q].wait()
                    return carry
                lax.fori_loop(0, nit, body, 0)

        @pl.when(c == 0)
        def _():
            run(zlo_hbm)

        @pl.when(c == 1)
        def _():
            run(zhi_hbm)

        plsc.subcore_barrier()

        def wout(out_hbm):
            for kb in range(ROWS_PER_TILE // 128):
                base = s * ROWS_PER_TILE + kb * 128
                pltpu.sync_copy(acc.at[pl.ds(base, 64)], b0)
                pltpu.sync_copy(acc.at[pl.ds(base + 64, 64)], b1)
                pltpu.sync_copy(b0, out_hbm.at[pl.ds(base, 64)])
                pltpu.sync_copy(b1, out_hbm.at[pl.ds(base + 64, 64)])

        @pl.when(c == 0)
        def _():
            wout(outlo)

        @pl.when(c == 1)
        def _():
            wout(outhi)

    return k(zlo, zhi, src2d, dst2d)


_RB = 1024
_GRID = NPAD // _RB


def _tc_matmul(x_pad, W):
    def body(x_ref, w_ref, o_ref):
        o_ref[...] = jnp.dot(x_ref[...], w_ref[...],
                             preferred_element_type=jnp.float32)
    return pl.pallas_call(
        body,
        grid=(_GRID,),
        in_specs=[pl.BlockSpec((_RB, D), lambda i: (i, 0)),
                  pl.BlockSpec((D, D), lambda i: (0, 0))],
        out_specs=pl.BlockSpec((_RB, D), lambda i: (i, 0)),
        out_shape=jax.ShapeDtypeStruct((NPAD, D), jnp.float32),
    )(x_pad, W)


def _tc_scale(xw, dega, degb):
    def body(xw_ref, da_ref, db_ref, zlo_ref, zhi_ref, dinv_ref):
        d = da_ref[:, 0:1] + db_ref[:, 0:1] + 1.0
        dinv = jnp.where(d > 0, lax.rsqrt(d), 0.0)
        xwv = xw_ref[...]
        zlo_ref[...] = xwv[:, :DH] * dinv
        zhi_ref[...] = xwv[:, DH:] * dinv
        dinv_ref[...] = dinv
    return pl.pallas_call(
        body,
        grid=(_GRID,),
        in_specs=[pl.BlockSpec((_RB, D), lambda i: (i, 0)),
                  pl.BlockSpec((_RB, 128), lambda i: (i, 0)),
                  pl.BlockSpec((_RB, 128), lambda i: (i, 0))],
        out_specs=[pl.BlockSpec((_RB, DH), lambda i: (i, 0)),
                   pl.BlockSpec((_RB, DH), lambda i: (i, 0)),
                   pl.BlockSpec((_RB, 1), lambda i: (i, 0))],
        out_shape=(jax.ShapeDtypeStruct((NPAD, DH), jnp.float32),
                   jax.ShapeDtypeStruct((NPAD, DH), jnp.float32),
                   jax.ShapeDtypeStruct((NPAD, 1), jnp.float32)),
    )(xw, dega, degb)


def _tc_layer2(acclo, acchi, xw1, dinv, b1r, W2):
    def body(alo_ref, ahi_ref, xw_ref, dv_ref, b_ref, w_ref,
             xw2_ref, zlo_ref, zhi_ref):
        dv = dv_ref[...]
        accv = jnp.concatenate([alo_ref[...], ahi_ref[...]], axis=1)
        h = jnp.maximum(dv * accv + (dv * dv) * xw_ref[...] + b_ref[...], 0.0)
        xw2 = jnp.dot(h, w_ref[...], preferred_element_type=jnp.float32)
        xw2_ref[...] = xw2
        zlo_ref[...] = xw2[:, :DH] * dv
        zhi_ref[...] = xw2[:, DH:] * dv
    return pl.pallas_call(
        body,
        grid=(_GRID,),
        in_specs=[pl.BlockSpec((_RB, DH), lambda i: (i, 0)),
                  pl.BlockSpec((_RB, DH), lambda i: (i, 0)),
                  pl.BlockSpec((_RB, D), lambda i: (i, 0)),
                  pl.BlockSpec((_RB, 1), lambda i: (i, 0)),
                  pl.BlockSpec((1, D), lambda i: (0, 0)),
                  pl.BlockSpec((D, D), lambda i: (0, 0))],
        out_specs=[pl.BlockSpec((_RB, D), lambda i: (i, 0)),
                   pl.BlockSpec((_RB, DH), lambda i: (i, 0)),
                   pl.BlockSpec((_RB, DH), lambda i: (i, 0))],
        out_shape=(jax.ShapeDtypeStruct((NPAD, D), jnp.float32),
                   jax.ShapeDtypeStruct((NPAD, DH), jnp.float32),
                   jax.ShapeDtypeStruct((NPAD, DH), jnp.float32)),
    )(acclo, acchi, xw1, dinv, b1r, W2)


def _tc_final(acclo, acchi, xw2, dinv, b2r, fc_w, fc_b8):
    def body(alo_ref, ahi_ref, xw_ref, dv_ref, b_ref, fw_ref, fb_ref, o_ref):
        i = pl.program_id(0)
        dv = dv_ref[...]
        accv = jnp.concatenate([alo_ref[...], ahi_ref[...]], axis=1)
        h = jnp.maximum(dv * accv + (dv * dv) * xw_ref[...] + b_ref[...], 0.0)
        row = lax.broadcasted_iota(jnp.int32, (_RB, 1), 0) + i * _RB
        h = jnp.where(row < N, h, 0.0)
        ps = jnp.sum(h * fw_ref[...])

        @pl.when(i == 0)
        def _():
            o_ref[...] = fb_ref[...]
        o_ref[...] = o_ref[...] + ps * (1.0 / N)
    return pl.pallas_call(
        body,
        grid=(_GRID,),
        in_specs=[pl.BlockSpec((_RB, DH), lambda i: (i, 0)),
                  pl.BlockSpec((_RB, DH), lambda i: (i, 0)),
                  pl.BlockSpec((_RB, D), lambda i: (i, 0)),
                  pl.BlockSpec((_RB, 1), lambda i: (i, 0)),
                  pl.BlockSpec((1, D), lambda i: (0, 0)),
                  pl.BlockSpec((1, D), lambda i: (0, 0)),
                  pl.BlockSpec((8, 128), lambda i: (0, 0))],
        out_specs=pl.BlockSpec((8, 128), lambda i: (0, 0)),
        out_shape=jax.ShapeDtypeStruct((8, 128), jnp.float32),
    )(acclo, acchi, xw2, dinv, b2r, fc_w, fc_b8)


def kernel(x, edge_index, edge_attr, W1, b1, W2, b2, fc_w, fc_b):
    src = edge_index[0].astype(jnp.int32)
    dst = edge_index[1].astype(jnp.int32)

    padn = EPAD - E
    ar = jnp.arange(padn, dtype=jnp.int32)
    src_f = jnp.concatenate([src, N + (ar % (NPAD - N))])
    dst_f = jnp.concatenate([dst, N + ((ar + 97) % (NPAD - N))])
    dst_p = dst_f.reshape(NCHUNK, 128)
    src_p64 = src_f.reshape(NCHUNK2, 64)
    dst_p64 = dst_f.reshape(NCHUNK2, 64)

    x_pad = jnp.pad(x, ((0, NPAD - N), (0, 0)))
    b1r = b1.reshape(1, D)
    b2r = b2.reshape(1, D)
    fc_b8 = jnp.broadcast_to(fc_b.reshape(1, 1), (8, 128))

    dega, degb = _sc_degree(dst_p)
    xw1 = _tc_matmul(x_pad, W1)
    z1lo, z1hi, dinv = _tc_scale(xw1, dega, degb)
    acc1lo, acc1hi = _sc_agg(z1lo, z1hi, src_p64, dst_p64)
    xw2, z2lo, z2hi = _tc_layer2(acc1lo, acc1hi, xw1, dinv, b1r, W2)
    acc2lo, acc2hi = _sc_agg(z2lo, z2hi, src_p64, dst_p64)
    out8 = _tc_final(acc2lo, acc2hi, xw2, dinv, b2r, fc_w, fc_b8)
    return jnp.reshape(out8[0, 0], (1,))


# speedup vs baseline: 1.0050x; 1.0050x over previous
import functools

import jax
import jax.numpy as jnp
from jax import lax
from jax.experimental import pallas as pl
from jax.experimental.pallas import tpu as pltpu
from jax.experimental.pallas import tpu_sc as plsc

N = 10000
NPAD = 10240
D = 256
DH = 128
E = 160000
EPAD = 163840
NCHUNK = EPAD // 128
NCHUNK2 = EPAD // 64
KA2 = NCHUNK2 // 16
KD = NCHUNK // 32
ROWS_PER_TILE = NPAD // 16

_MESH = plsc.VectorSubcoreMesh(
    core_axis_name="c", subcore_axis_name="s", num_cores=2, num_subcores=16)


def _sc_degree(dst2d):
    @functools.partial(
        pl.kernel,
        out_type=(jax.ShapeDtypeStruct((NPAD, 128), jnp.float32),
                  jax.ShapeDtypeStruct((NPAD, 128), jnp.float32)),
        mesh=_MESH,
        scratch_types=[
            pltpu.VMEM((KD, 128), jnp.int32),
            pltpu.VMEM((128, 128), jnp.float32),
            pltpu.VMEM((128, 128), jnp.float32),
            pltpu.VMEM_SHARED((NPAD, 128), jnp.float32),
        ],
    )
    def k(dst_hbm, dega_hbm, degb_hbm, dstv, ones_v, buf, acc):
        c = lax.axis_index("c")
        s = lax.axis_index("s")
        zero16 = jnp.zeros((16,), jnp.float32)
        one16 = jnp.ones((16,), jnp.float32)

        def zrow(r, carry):
            for g in range(8):
                buf[r, pl.ds(g * 16, 16)] = zero16
            return carry
        lax.fori_loop(0, 128, zrow, 0)

        def orow(r, carry):
            for g in range(8):
                ones_v[r, pl.ds(g * 16, 16)] = one16
            return carry
        lax.fori_loop(0, 128, orow, 0)

        for kb in range(ROWS_PER_TILE // 128):
            pltpu.sync_copy(buf, acc.at[pl.ds(s * ROWS_PER_TILE + kb * 128, 128)])
        plsc.subcore_barrier()

        pltpu.sync_copy(dst_hbm.at[pl.ds((c * 16 + s) * KD, KD)], dstv)

        def body(j, carry):
            pltpu.sync_copy(ones_v, acc.at[dstv.at[j]], add=True)
            return carry
        lax.fori_loop(0, KD, body, 0)

        plsc.subcore_barrier()

        def wout(out_hbm):
            for kb in range(ROWS_PER_TILE // 128):
                base = s * ROWS_PER_TILE + kb * 128
                pltpu.sync_copy(acc.at[pl.ds(base, 128)], buf)
                pltpu.sync_copy(buf, out_hbm.at[pl.ds(base, 128)])

        @pl.when(c == 0)
        def _():
            wout(dega_hbm)

        @pl.when(c == 1)
        def _():
            wout(degb_hbm)

    return k(dst2d)


KA = NCHUNK // 16


def _sc_agg(zlo, zhi, src2d, dst2d):
    @functools.partial(
        pl.kernel,
        out_type=(jax.ShapeDtypeStruct((NPAD, DH), jnp.float32),
                  jax.ShapeDtypeStruct((NPAD, DH), jnp.float32)),
        mesh=_MESH,
        scratch_types=[
            pltpu.VMEM((KA // 2, 128), jnp.int32),
            pltpu.VMEM((KA // 2, 128), jnp.int32),
            pltpu.VMEM((128, DH), jnp.float32),
            pltpu.VMEM((128, DH), jnp.float32),
            pltpu.VMEM_SHARED((NPAD, DH), jnp.float32),
            pltpu.SemaphoreType.DMA,
            pltpu.SemaphoreType.DMA,
        ],
    )
    def k(zlo_hbm, zhi_hbm, src_hbm, dst_hbm, outlo, outhi,
          srcv, dstv, bufa, bufb, acc, sema, semb):
        c = lax.axis_index("c")
        s = lax.axis_index("s")
        zero16 = jnp.zeros((16,), jnp.float32)

        def zrow(r, carry):
            for g in range(DH // 16):
                bufa[r, pl.ds(g * 16, 16)] = zero16
            return carry
        lax.fori_loop(0, 128, zrow, 0)
        for kb in range(ROWS_PER_TILE // 128):
            pltpu.sync_copy(bufa, acc.at[pl.ds(s * ROWS_PER_TILE + kb * 128, 128)])
        plsc.subcore_barrier()

        def run(z_hbm):
            kh = KA // 2
            for half in range(2):
                pltpu.sync_copy(src_hbm.at[pl.ds(s * KA + half * kh, kh)], srcv)
                pltpu.sync_copy(dst_hbm.at[pl.ds(s * KA + half * kh, kh)], dstv)
                nit = kh // 2

                def body(jj, carry):
                    j0 = jj * 2
                    d0 = pltpu.async_copy(z_hbm.at[srcv.at[j0]], bufa, sema)
                    d1 = pltpu.async_copy(z_hbm.at[srcv.at[j0 + 1]], bufb, semb)
                    d0.wait()
                    pltpu.sync_copy(bufa, acc.at[dstv.at[j0]], add=True)
                    d1.wait()
                    pltpu.sync_copy(bufb, acc.at[dstv.at[j0 + 1]], add=True)
                    return carry
                lax.fori_loop(0, nit, body, 0)

        @pl.when(c == 0)
        def _():
            run(zlo_hbm)

        @pl.when(c == 1)
        def _():
            run(zhi_hbm)

        plsc.subcore_barrier()

        def wout(out_hbm):
            for kb in range(ROWS_PER_TILE // 128):
                base = s * ROWS_PER_TILE + kb * 128
                pltpu.sync_copy(acc.at[pl.ds(base, 128)], bufa)
                pltpu.sync_copy(bufa, out_hbm.at[pl.ds(base, 128)])

        @pl.when(c == 0)
        def _():
            wout(outlo)

        @pl.when(c == 1)
        def _():
            wout(outhi)

    return k(zlo, zhi, src2d, dst2d)


_RB = 1024
_GRID = NPAD // _RB


def _tc_matmul_scale(x_pad, W, dega, degb):
    def body(x_ref, w_ref, da_ref, db_ref, xw_ref, zlo_ref, zhi_ref, dinv_ref):
        d = da_ref[:, 0:1] + db_ref[:, 0:1] + 1.0
        dinv = jnp.where(d > 0, lax.rsqrt(d), 0.0)
        xw = jnp.dot(x_ref[...], w_ref[...], preferred_element_type=jnp.float32)
        xw_ref[...] = xw
        zlo_ref[...] = xw[:, :DH] * dinv
        zhi_ref[...] = xw[:, DH:] * dinv
        dinv_ref[...] = dinv
    return pl.pallas_call(
        body,
        grid=(_GRID,),
        in_specs=[pl.BlockSpec((_RB, D), lambda i: (i, 0)),
                  pl.BlockSpec((D, D), lambda i: (0, 0)),
                  pl.BlockSpec((_RB, 128), lambda i: (i, 0)),
                  pl.BlockSpec((_RB, 128), lambda i: (i, 0))],
        out_specs=[pl.BlockSpec((_RB, D), lambda i: (i, 0)),
                   pl.BlockSpec((_RB, DH), lambda i: (i, 0)),
                   pl.BlockSpec((_RB, DH), lambda i: (i, 0)),
                   pl.BlockSpec((_RB, 1), lambda i: (i, 0))],
        out_shape=(jax.ShapeDtypeStruct((NPAD, D), jnp.float32),
                   jax.ShapeDtypeStruct((NPAD, DH), jnp.float32),
                   jax.ShapeDtypeStruct((NPAD, DH), jnp.float32),
                   jax.ShapeDtypeStruct((NPAD, 1), jnp.float32)),
    )(x_pad, W, dega, degb)


def _tc_layer2(acclo, acchi, xw1, dinv, b1r, W2):
    def body(alo_ref, ahi_ref, xw_ref, dv_ref, b_ref, w_ref,
             xw2_ref, zlo_ref, zhi_ref):
        dv = dv_ref[...]
        accv = jnp.concatenate([alo_ref[...], ahi_ref[...]], axis=1)
        h = jnp.maximum(dv * accv + (dv * dv) * xw_ref[...] + b_ref[...], 0.0)
        xw2 = jnp.dot(h, w_ref[...], preferred_element_type=jnp.float32)
        xw2_ref[...] = xw2
        zlo_ref[...] = xw2[:, :DH] * dv
        zhi_ref[...] = xw2[:, DH:] * dv
    return pl.pallas_call(
        body,
        grid=(_GRID,),
        in_specs=[pl.BlockSpec((_RB, DH), lambda i: (i, 0)),
                  pl.BlockSpec((_RB, DH), lambda i: (i, 0)),
                  pl.BlockSpec((_RB, D), lambda i: (i, 0)),
                  pl.BlockSpec((_RB, 1), lambda i: (i, 0)),
                  pl.BlockSpec((1, D), lambda i: (0, 0)),
                  pl.BlockSpec((D, D), lambda i: (0, 0))],
        out_specs=[pl.BlockSpec((_RB, D), lambda i: (i, 0)),
                   pl.BlockSpec((_RB, DH), lambda i: (i, 0)),
                   pl.BlockSpec((_RB, DH), lambda i: (i, 0))],
        out_shape=(jax.ShapeDtypeStruct((NPAD, D), jnp.float32),
                   jax.ShapeDtypeStruct((NPAD, DH), jnp.float32),
                   jax.ShapeDtypeStruct((NPAD, DH), jnp.float32)),
    )(acclo, acchi, xw1, dinv, b1r, W2)


def _tc_final(acclo, acchi, xw2, dinv, b2r, fc_w, fc_b8):
    def body(alo_ref, ahi_ref, xw_ref, dv_ref, b_ref, fw_ref, fb_ref, o_ref):
        i = pl.program_id(0)
        dv = dv_ref[...]
        accv = jnp.concatenate([alo_ref[...], ahi_ref[...]], axis=1)
        h = jnp.maximum(dv * accv + (dv * dv) * xw_ref[...] + b_ref[...], 0.0)
        row = lax.broadcasted_iota(jnp.int32, (_RB, 1), 0) + i * _RB
        h = jnp.where(row < N, h, 0.0)
        ps = jnp.sum(h * fw_ref[...])

        @pl.when(i == 0)
        def _():
            o_ref[...] = fb_ref[...]
        o_ref[...] = o_ref[...] + ps * (1.0 / N)
    return pl.pallas_call(
        body,
        grid=(_GRID,),
        in_specs=[pl.BlockSpec((_RB, DH), lambda i: (i, 0)),
                  pl.BlockSpec((_RB, DH), lambda i: (i, 0)),
                  pl.BlockSpec((_RB, D), lambda i: (i, 0)),
                  pl.BlockSpec((_RB, 1), lambda i: (i, 0)),
                  pl.BlockSpec((1, D), lambda i: (0, 0)),
                  pl.BlockSpec((1, D), lambda i: (0, 0)),
                  pl.BlockSpec((8, 128), lambda i: (0, 0))],
        out_specs=pl.BlockSpec((8, 128), lambda i: (0, 0)),
        out_shape=jax.ShapeDtypeStruct((8, 128), jnp.float32),
    )(acclo, acchi, xw2, dinv, b2r, fc_w, fc_b8)


def kernel(x, edge_index, edge_attr, W1, b1, W2, b2, fc_w, fc_b):
    src = edge_index[0].astype(jnp.int32)
    dst = edge_index[1].astype(jnp.int32)

    padn = EPAD - E
    ar = jnp.arange(padn, dtype=jnp.int32)
    src_f = jnp.concatenate([src, N + (ar % (NPAD - N))])
    dst_f = jnp.concatenate([dst, N + ((ar + 97) % (NPAD - N))])
    src_p = src_f.reshape(NCHUNK, 128)
    dst_p = dst_f.reshape(NCHUNK, 128)

    x_pad = jnp.pad(x, ((0, NPAD - N), (0, 0)))
    b1r = b1.reshape(1, D)
    b2r = b2.reshape(1, D)
    fc_b8 = jnp.broadcast_to(fc_b.reshape(1, 1), (8, 128))

    dega, degb = _sc_degree(dst_p)
    xw1, z1lo, z1hi, dinv = _tc_matmul_scale(x_pad, W1, dega, degb)
    acc1lo, acc1hi = _sc_agg(z1lo, z1hi, src_p, dst_p)
    xw2, z2lo, z2hi = _tc_layer2(acc1lo, acc1hi, xw1, dinv, b1r, W2)
    acc2lo, acc2hi = _sc_agg(z2lo, z2hi, src_p, dst_p)
    out8 = _tc_final(acc2lo, acc2hi, xw2, dinv, b2r, fc_w, fc_b8)
    return jnp.reshape(out8[0, 0], (1,))


# speedup vs baseline: 1.0205x; 1.0154x over previous
import functools

import jax
import jax.numpy as jnp
from jax import lax
from jax.experimental import pallas as pl
from jax.experimental.pallas import tpu as pltpu
from jax.experimental.pallas import tpu_sc as plsc

N = 10000
NPAD = 10240
D = 256
DH = 128
E = 160000
EPAD = 163840
NCHUNK = EPAD // 128
NCHUNK2 = EPAD // 64
KA2 = NCHUNK2 // 16
KD = NCHUNK // 32
ROWS_PER_TILE = NPAD // 16

_MESH = plsc.VectorSubcoreMesh(
    core_axis_name="c", subcore_axis_name="s", num_cores=2, num_subcores=16)


def _sc_degree(dst2d):
    @functools.partial(
        pl.kernel,
        out_type=(jax.ShapeDtypeStruct((NPAD, 128), jnp.float32),
                  jax.ShapeDtypeStruct((NPAD, 128), jnp.float32)),
        mesh=_MESH,
        scratch_types=[
            pltpu.VMEM((KD, 128), jnp.int32),
            pltpu.VMEM((128, 128), jnp.float32),
            pltpu.VMEM((128, 128), jnp.float32),
            pltpu.VMEM_SHARED((NPAD, 128), jnp.float32),
        ],
    )
    def k(dst_hbm, dega_hbm, degb_hbm, dstv, ones_v, buf, acc):
        c = lax.axis_index("c")
        s = lax.axis_index("s")
        zero16 = jnp.zeros((16,), jnp.float32)
        one16 = jnp.ones((16,), jnp.float32)

        def zrow(r, carry):
            for g in range(8):
                buf[r, pl.ds(g * 16, 16)] = zero16
            return carry
        lax.fori_loop(0, 128, zrow, 0)

        def orow(r, carry):
            for g in range(8):
                ones_v[r, pl.ds(g * 16, 16)] = one16
            return carry
        lax.fori_loop(0, 128, orow, 0)

        for kb in range(ROWS_PER_TILE // 128):
            pltpu.sync_copy(buf, acc.at[pl.ds(s * ROWS_PER_TILE + kb * 128, 128)])
        plsc.subcore_barrier()

        pltpu.sync_copy(dst_hbm.at[pl.ds((c * 16 + s) * KD, KD)], dstv)

        def body(j, carry):
            pltpu.sync_copy(ones_v, acc.at[dstv.at[j]], add=True)
            return carry
        lax.fori_loop(0, KD, body, 0)

        plsc.subcore_barrier()

        def wout(out_hbm):
            for kb in range(ROWS_PER_TILE // 128):
                base = s * ROWS_PER_TILE + kb * 128
                pltpu.sync_copy(acc.at[pl.ds(base, 128)], buf)
                pltpu.sync_copy(buf, out_hbm.at[pl.ds(base, 128)])

        @pl.when(c == 0)
        def _():
            wout(dega_hbm)

        @pl.when(c == 1)
        def _():
            wout(degb_hbm)

    return k(dst2d)


KA = NCHUNK // 16


def _sc_agg(zlo, zhi, src2d, dst2d):
    @functools.partial(
        pl.kernel,
        out_type=(jax.ShapeDtypeStruct((NPAD, DH), jnp.float32),
                  jax.ShapeDtypeStruct((NPAD, DH), jnp.float32)),
        mesh=_MESH,
        scratch_types=[
            pltpu.VMEM((KA // 2, 128), jnp.int32),
            pltpu.VMEM((KA // 2, 128), jnp.int32),
            pltpu.VMEM((128, DH), jnp.float32),
            pltpu.VMEM((128, DH), jnp.float32),
            pltpu.VMEM_SHARED((NPAD, DH), jnp.float32),
            pltpu.SemaphoreType.DMA,
            pltpu.SemaphoreType.DMA,
        ],
    )
    def k(zlo_hbm, zhi_hbm, src_hbm, dst_hbm, outlo, outhi,
          srcv, dstv, bufa, bufb, acc, sema, semb):
        c = lax.axis_index("c")
        s = lax.axis_index("s")
        zero16 = jnp.zeros((16,), jnp.float32)
        kh = KA // 2

        da = pltpu.async_copy(src_hbm.at[pl.ds(s * KA, kh)], srcv, sema)
        db = pltpu.async_copy(dst_hbm.at[pl.ds(s * KA, kh)], dstv, semb)

        def zrow(r, carry):
            for g in range(DH // 16):
                bufa[r, pl.ds(g * 16, 16)] = zero16
            return carry
        lax.fori_loop(0, 128, zrow, 0)
        for kb in range(ROWS_PER_TILE // 128):
            pltpu.sync_copy(bufa, acc.at[pl.ds(s * ROWS_PER_TILE + kb * 128, 128)])
        da.wait()
        db.wait()
        plsc.subcore_barrier()

        def run(z_hbm):
            for half in range(2):
                if half > 0:
                    pltpu.sync_copy(src_hbm.at[pl.ds(s * KA + half * kh, kh)], srcv)
                    pltpu.sync_copy(dst_hbm.at[pl.ds(s * KA + half * kh, kh)], dstv)
                nit = kh // 2

                def body(jj, carry):
                    j0 = jj * 2
                    d0 = pltpu.async_copy(z_hbm.at[srcv.at[j0]], bufa, sema)
                    d1 = pltpu.async_copy(z_hbm.at[srcv.at[j0 + 1]], bufb, semb)
                    d0.wait()
                    pltpu.sync_copy(bufa, acc.at[dstv.at[j0]], add=True)
                    d1.wait()
                    pltpu.sync_copy(bufb, acc.at[dstv.at[j0 + 1]], add=True)
                    return carry
                lax.fori_loop(0, nit, body, 0)

        @pl.when(c == 0)
        def _():
            run(zlo_hbm)

        @pl.when(c == 1)
        def _():
            run(zhi_hbm)

        plsc.subcore_barrier()

        def wout(out_hbm):
            base = s * ROWS_PER_TILE
            pltpu.sync_copy(acc.at[pl.ds(base, ROWS_PER_TILE)],
                            out_hbm.at[pl.ds(base, ROWS_PER_TILE)])

        @pl.when(c == 0)
        def _():
            wout(outlo)

        @pl.when(c == 1)
        def _():
            wout(outhi)

    return k(zlo, zhi, src2d, dst2d)


_RB = 1024
_GRID = NPAD // _RB


def _tc_matmul(x_pad, W):
    def body(x_ref, w_ref, o_ref):
        o_ref[...] = jnp.dot(x_ref[...], w_ref[...],
                             preferred_element_type=jnp.float32)
    return pl.pallas_call(
        body,
        grid=(_GRID,),
        in_specs=[pl.BlockSpec((_RB, D), lambda i: (i, 0)),
                  pl.BlockSpec((D, D), lambda i: (0, 0))],
        out_specs=pl.BlockSpec((_RB, D), lambda i: (i, 0)),
        out_shape=jax.ShapeDtypeStruct((NPAD, D), jnp.float32),
    )(x_pad, W)


def _tc_scale(xw, dega, degb):
    def body(xw_ref, da_ref, db_ref, zlo_ref, zhi_ref, dinv_ref):
        d = da_ref[:, 0:1] + db_ref[:, 0:1] + 1.0
        dinv = jnp.where(d > 0, lax.rsqrt(d), 0.0)
        xwv = xw_ref[...]
        zlo_ref[...] = xwv[:, :DH] * dinv
        zhi_ref[...] = xwv[:, DH:] * dinv
        dinv_ref[...] = dinv
    return pl.pallas_call(
        body,
        grid=(_GRID,),
        in_specs=[pl.BlockSpec((_RB, D), lambda i: (i, 0)),
                  pl.BlockSpec((_RB, 128), lambda i: (i, 0)),
                  pl.BlockSpec((_RB, 128), lambda i: (i, 0))],
        out_specs=[pl.BlockSpec((_RB, DH), lambda i: (i, 0)),
                   pl.BlockSpec((_RB, DH), lambda i: (i, 0)),
                   pl.BlockSpec((_RB, 1), lambda i: (i, 0))],
        out_shape=(jax.ShapeDtypeStruct((NPAD, DH), jnp.float32),
                   jax.ShapeDtypeStruct((NPAD, DH), jnp.float32),
                   jax.ShapeDtypeStruct((NPAD, 1), jnp.float32)),
    )(xw, dega, degb)


def _tc_layer2(acclo, acchi, xw1, dinv, b1r, W2):
    def body(alo_ref, ahi_ref, xw_ref, dv_ref, b_ref, w_ref,
             xw2_ref, zlo_ref, zhi_ref):
        dv = dv_ref[...]
        accv = jnp.concatenate([alo_ref[...], ahi_ref[...]], axis=1)
        h = jnp.maximum(dv * accv + (dv * dv) * xw_ref[...] + b_ref[...], 0.0)
        xw2 = jnp.dot(h, w_ref[...], preferred_element_type=jnp.float32)
        xw2_ref[...] = xw2
        zlo_ref[...] = xw2[:, :DH] * dv
        zhi_ref[...] = xw2[:, DH:] * dv
    return pl.pallas_call(
        body,
        grid=(_GRID,),
        in_specs=[pl.BlockSpec((_RB, DH), lambda i: (i, 0)),
                  pl.BlockSpec((_RB, DH), lambda i: (i, 0)),
                  pl.BlockSpec((_RB, D), lambda i: (i, 0)),
                  pl.BlockSpec((_RB, 1), lambda i: (i, 0)),
                  pl.BlockSpec((1, D), lambda i: (0, 0)),
                  pl.BlockSpec((D, D), lambda i: (0, 0))],
        out_specs=[pl.BlockSpec((_RB, D), lambda i: (i, 0)),
                   pl.BlockSpec((_RB, DH), lambda i: (i, 0)),
                   pl.BlockSpec((_RB, DH), lambda i: (i, 0))],
        out_shape=(jax.ShapeDtypeStruct((NPAD, D), jnp.float32),
                   jax.ShapeDtypeStruct((NPAD, DH), jnp.float32),
                   jax.ShapeDtypeStruct((NPAD, DH), jnp.float32)),
    )(acclo, acchi, xw1, dinv, b1r, W2)


def _tc_final(acclo, acchi, xw2, dinv, b2r, fc_w, fc_b8):
    def body(alo_ref, ahi_ref, xw_ref, dv_ref, b_ref, fw_ref, fb_ref, o_ref):
        i = pl.program_id(0)
        dv = dv_ref[...]
        accv = jnp.concatenate([alo_ref[...], ahi_ref[...]], axis=1)
        h = jnp.maximum(dv * accv + (dv * dv) * xw_ref[...] + b_ref[...], 0.0)
        row = lax.broadcasted_iota(jnp.int32, (_RB, 1), 0) + i * _RB
        h = jnp.where(row < N, h, 0.0)
        ps = jnp.sum(h * fw_ref[...])

        @pl.when(i == 0)
        def _():
            o_ref[...] = fb_ref[...]
        o_ref[...] = o_ref[...] + ps * (1.0 / N)
    return pl.pallas_call(
        body,
        grid=(_GRID,),
        in_specs=[pl.BlockSpec((_RB, DH), lambda i: (i, 0)),
                  pl.BlockSpec((_RB, DH), lambda i: (i, 0)),
                  pl.BlockSpec((_RB, D), lambda i: (i, 0)),
                  pl.BlockSpec((_RB, 1), lambda i: (i, 0)),
                  pl.BlockSpec((1, D), lambda i: (0, 0)),
                  pl.BlockSpec((1, D), lambda i: (0, 0)),
                  pl.BlockSpec((8, 128), lambda i: (0, 0))],
        out_specs=pl.BlockSpec((8, 128), lambda i: (0, 0)),
        out_shape=jax.ShapeDtypeStruct((8, 128), jnp.float32),
    )(acclo, acchi, xw2, dinv, b2r, fc_w, fc_b8)


def kernel(x, edge_index, edge_attr, W1, b1, W2, b2, fc_w, fc_b):
    src = edge_index[0].astype(jnp.int32)
    dst = edge_index[1].astype(jnp.int32)

    padn = EPAD - E
    ar = jnp.arange(padn, dtype=jnp.int32)
    src_f = jnp.concatenate([src, N + (ar % (NPAD - N))])
    dst_f = jnp.concatenate([dst, N + ((ar + 97) % (NPAD - N))])
    src_p = src_f.reshape(NCHUNK, 128)
    dst_p = dst_f.reshape(NCHUNK, 128)

    x_pad = jnp.pad(x, ((0, NPAD - N), (0, 0)))
    b1r = b1.reshape(1, D)
    b2r = b2.reshape(1, D)
    fc_b8 = jnp.broadcast_to(fc_b.reshape(1, 1), (8, 128))

    dega, degb = _sc_degree(dst_p)
    xw1 = _tc_matmul(x_pad, W1)
    z1lo, z1hi, dinv = _tc_scale(xw1, dega, degb)
    acc1lo, acc1hi = _sc_agg(z1lo, z1hi, src_p, dst_p)
    xw2, z2lo, z2hi = _tc_layer2(acc1lo, acc1hi, xw1, dinv, b1r, W2)
    acc2lo, acc2hi = _sc_agg(z2lo, z2hi, src_p, dst_p)
    out8 = _tc_final(acc2lo, acc2hi, xw2, dinv, b2r, fc_w, fc_b8)
    return jnp.reshape(out8[0, 0], (1,))


# speedup vs baseline: 1.0254x; 1.0049x over previous
import functools

import jax
import jax.numpy as jnp
from jax import lax
from jax.experimental import pallas as pl
from jax.experimental.pallas import tpu as pltpu
from jax.experimental.pallas import tpu_sc as plsc

N = 10000
NPAD = 10240
D = 256
DH = 128
E = 160000
EPAD = 163840
NCHUNK = EPAD // 128
NCHUNK2 = EPAD // 64
KA2 = NCHUNK2 // 16
KD = NCHUNK // 32
ROWS_PER_TILE = NPAD // 16

_MESH = plsc.VectorSubcoreMesh(
    core_axis_name="c", subcore_axis_name="s", num_cores=2, num_subcores=16)


def _sc_degree(dst2d):
    @functools.partial(
        pl.kernel,
        out_type=(jax.ShapeDtypeStruct((NPAD, 128), jnp.float32),
                  jax.ShapeDtypeStruct((NPAD, 128), jnp.float32)),
        mesh=_MESH,
        scratch_types=[
            pltpu.VMEM((KD, 128), jnp.int32),
            pltpu.VMEM((128, 128), jnp.float32),
            pltpu.VMEM((128, 128), jnp.float32),
            pltpu.VMEM_SHARED((NPAD, 128), jnp.float32),
            pltpu.SemaphoreType.DMA,
        ],
    )
    def k(dst_hbm, dega_hbm, degb_hbm, dstv, ones_v, buf, acc, sem):
        c = lax.axis_index("c")
        s = lax.axis_index("s")
        zero16 = jnp.zeros((16,), jnp.float32)
        one16 = jnp.ones((16,), jnp.float32)

        di = pltpu.async_copy(dst_hbm.at[pl.ds((c * 16 + s) * KD, KD)], dstv, sem)

        def zrow(r, carry):
            for g in range(8):
                buf[r, pl.ds(g * 16, 16)] = zero16
                ones_v[r, pl.ds(g * 16, 16)] = one16
            return carry
        lax.fori_loop(0, 128, zrow, 0)

        for kb in range(ROWS_PER_TILE // 128):
            pltpu.sync_copy(buf, acc.at[pl.ds(s * ROWS_PER_TILE + kb * 128, 128)])
        di.wait()
        plsc.subcore_barrier()

        def body(j, carry):
            pltpu.sync_copy(ones_v, acc.at[dstv.at[j]], add=True)
            return carry
        lax.fori_loop(0, KD, body, 0)

        plsc.subcore_barrier()

        def wout(out_hbm):
            base = s * ROWS_PER_TILE
            pltpu.sync_copy(acc.at[pl.ds(base, ROWS_PER_TILE)],
                            out_hbm.at[pl.ds(base, ROWS_PER_TILE)])

        @pl.when(c == 0)
        def _():
            wout(dega_hbm)

        @pl.when(c == 1)
        def _():
            wout(degb_hbm)

    return k(dst2d)


KA = NCHUNK // 16


def _sc_agg(zlo, zhi, src2d, dst2d):
    @functools.partial(
        pl.kernel,
        out_type=(jax.ShapeDtypeStruct((NPAD, DH), jnp.float32),
                  jax.ShapeDtypeStruct((NPAD, DH), jnp.float32)),
        mesh=_MESH,
        scratch_types=[
            pltpu.VMEM((KA // 2, 128), jnp.int32),
            pltpu.VMEM((KA // 2, 128), jnp.int32),
            pltpu.VMEM((128, DH), jnp.float32),
            pltpu.VMEM((128, DH), jnp.float32),
            pltpu.VMEM_SHARED((NPAD, DH), jnp.float32),
            pltpu.SemaphoreType.DMA,
            pltpu.SemaphoreType.DMA,
        ],
    )
    def k(zlo_hbm, zhi_hbm, src_hbm, dst_hbm, outlo, outhi,
          srcv, dstv, bufa, bufb, acc, sema, semb):
        c = lax.axis_index("c")
        s = lax.axis_index("s")
        zero16 = jnp.zeros((16,), jnp.float32)
        kh = KA // 2

        da = pltpu.async_copy(src_hbm.at[pl.ds(s * KA, kh)], srcv, sema)
        db = pltpu.async_copy(dst_hbm.at[pl.ds(s * KA, kh)], dstv, semb)

        def zrow(r, carry):
            for g in range(DH // 16):
                bufa[r, pl.ds(g * 16, 16)] = zero16
            return carry
        lax.fori_loop(0, 128, zrow, 0)
        for kb in range(ROWS_PER_TILE // 128):
            pltpu.sync_copy(bufa, acc.at[pl.ds(s * ROWS_PER_TILE + kb * 128, 128)])
        da.wait()
        db.wait()
        plsc.subcore_barrier()

        def run(z_hbm):
            for half in range(2):
                if half > 0:
                    pltpu.sync_copy(src_hbm.at[pl.ds(s * KA + half * kh, kh)], srcv)
                    pltpu.sync_copy(dst_hbm.at[pl.ds(s * KA + half * kh, kh)], dstv)
                nit = kh // 2

                def body(jj, carry):
                    j0 = jj * 2
                    d0 = pltpu.async_copy(z_hbm.at[srcv.at[j0]], bufa, sema)
                    d1 = pltpu.async_copy(z_hbm.at[srcv.at[j0 + 1]], bufb, semb)
                    d0.wait()
                    pltpu.sync_copy(bufa, acc.at[dstv.at[j0]], add=True)
                    d1.wait()
                    pltpu.sync_copy(bufb, acc.at[dstv.at[j0 + 1]], add=True)
                    return carry
                lax.fori_loop(0, nit, body, 0)

        @pl.when(c == 0)
        def _():
            run(zlo_hbm)

        @pl.when(c == 1)
        def _():
            run(zhi_hbm)

        plsc.subcore_barrier()

        def wout(out_hbm):
            base = s * ROWS_PER_TILE
            pltpu.sync_copy(acc.at[pl.ds(base, ROWS_PER_TILE)],
                            out_hbm.at[pl.ds(base, ROWS_PER_TILE)])

        @pl.when(c == 0)
        def _():
            wout(outlo)

        @pl.when(c == 1)
        def _():
            wout(outhi)

    return k(zlo, zhi, src2d, dst2d)


_RB = 1024
_GRID = NPAD // _RB


def _tc_matmul(x_pad, W):
    def body(x_ref, w_ref, o_ref):
        o_ref[...] = jnp.dot(x_ref[...], w_ref[...],
                             preferred_element_type=jnp.float32)
    return pl.pallas_call(
        body,
        grid=(_GRID,),
        in_specs=[pl.BlockSpec((_RB, D), lambda i: (i, 0)),
                  pl.BlockSpec((D, D), lambda i: (0, 0))],
        out_specs=pl.BlockSpec((_RB, D), lambda i: (i, 0)),
        out_shape=jax.ShapeDtypeStruct((NPAD, D), jnp.float32),
    )(x_pad, W)


def _tc_scale(xw, dega, degb):
    def body(xw_ref, da_ref, db_ref, zlo_ref, zhi_ref, dinv_ref):
        d = da_ref[:, 0:1] + db_ref[:, 0:1] + 1.0
        dinv = jnp.where(d > 0, lax.rsqrt(d), 0.0)
        xwv = xw_ref[...]
        zlo_ref[...] = xwv[:, :DH] * dinv
        zhi_ref[...] = xwv[:, DH:] * dinv
        dinv_ref[...] = dinv
    return pl.pallas_call(
        body,
        grid=(_GRID,),
        in_specs=[pl.BlockSpec((_RB, D), lambda i: (i, 0)),
                  pl.BlockSpec((_RB, 128), lambda i: (i, 0)),
                  pl.BlockSpec((_RB, 128), lambda i: (i, 0))],
        out_specs=[pl.BlockSpec((_RB, DH), lambda i: (i, 0)),
                   pl.BlockSpec((_RB, DH), lambda i: (i, 0)),
                   pl.BlockSpec((_RB, 1), lambda i: (i, 0))],
        out_shape=(jax.ShapeDtypeStruct((NPAD, DH), jnp.float32),
                   jax.ShapeDtypeStruct((NPAD, DH), jnp.float32),
                   jax.ShapeDtypeStruct((NPAD, 1), jnp.float32)),
    )(xw, dega, degb)


def _tc_layer2(acclo, acchi, xw1, dinv, b1r, W2):
    def body(alo_ref, ahi_ref, xw_ref, dv_ref, b_ref, w_ref,
             xw2_ref, zlo_ref, zhi_ref):
        dv = dv_ref[...]
        accv = jnp.concatenate([alo_ref[...], ahi_ref[...]], axis=1)
        h = jnp.maximum(dv * accv + (dv * dv) * xw_ref[...] + b_ref[...], 0.0)
        xw2 = jnp.dot(h, w_ref[...], preferred_element_type=jnp.float32)
        xw2_ref[...] = xw2
        zlo_ref[...] = xw2[:, :DH] * dv
        zhi_ref[...] = xw2[:, DH:] * dv
    return pl.pallas_call(
        body,
        grid=(_GRID,),
        in_specs=[pl.BlockSpec((_RB, DH), lambda i: (i, 0)),
                  pl.BlockSpec((_RB, DH), lambda i: (i, 0)),
                  pl.BlockSpec((_RB, D), lambda i: (i, 0)),
                  pl.BlockSpec((_RB, 1), lambda i: (i, 0)),
                  pl.BlockSpec((1, D), lambda i: (0, 0)),
                  pl.BlockSpec((D, D), lambda i: (0, 0))],
        out_specs=[pl.BlockSpec((_RB, D), lambda i: (i, 0)),
                   pl.BlockSpec((_RB, DH), lambda i: (i, 0)),
                   pl.BlockSpec((_RB, DH), lambda i: (i, 0))],
        out_shape=(jax.ShapeDtypeStruct((NPAD, D), jnp.float32),
                   jax.ShapeDtypeStruct((NPAD, DH), jnp.float32),
                   jax.ShapeDtypeStruct((NPAD, DH), jnp.float32)),
    )(acclo, acchi, xw1, dinv, b1r, W2)


def _tc_final(acclo, acchi, xw2, dinv, b2r, fc_w, fc_b8):
    def body(alo_ref, ahi_ref, xw_ref, dv_ref, b_ref, fw_ref, fb_ref, o_ref):
        i = pl.program_id(0)
        dv = dv_ref[...]
        accv = jnp.concatenate([alo_ref[...], ahi_ref[...]], axis=1)
        h = jnp.maximum(dv * accv + (dv * dv) * xw_ref[...] + b_ref[...], 0.0)
        row = lax.broadcasted_iota(jnp.int32, (_RB, 1), 0) + i * _RB
        h = jnp.where(row < N, h, 0.0)
        ps = jnp.sum(h * fw_ref[...])

        @pl.when(i == 0)
        def _():
            o_ref[...] = fb_ref[...]
        o_ref[...] = o_ref[...] + ps * (1.0 / N)
    return pl.pallas_call(
        body,
        grid=(_GRID,),
        in_specs=[pl.BlockSpec((_RB, DH), lambda i: (i, 0)),
                  pl.BlockSpec((_RB, DH), lambda i: (i, 0)),
                  pl.BlockSpec((_RB, D), lambda i: (i, 0)),
                  pl.BlockSpec((_RB, 1), lambda i: (i, 0)),
                  pl.BlockSpec((1, D), lambda i: (0, 0)),
                  pl.BlockSpec((1, D), lambda i: (0, 0)),
                  pl.BlockSpec((8, 128), lambda i: (0, 0))],
        out_specs=pl.BlockSpec((8, 128), lambda i: (0, 0)),
        out_shape=jax.ShapeDtypeStruct((8, 128), jnp.float32),
    )(acclo, acchi, xw2, dinv, b2r, fc_w, fc_b8)


def kernel(x, edge_index, edge_attr, W1, b1, W2, b2, fc_w, fc_b):
    src = edge_index[0].astype(jnp.int32)
    dst = edge_index[1].astype(jnp.int32)

    padn = EPAD - E
    ar = jnp.arange(padn, dtype=jnp.int32)
    src_f = jnp.concatenate([src, N + (ar % (NPAD - N))])
    dst_f = jnp.concatenate([dst, N + ((ar + 97) % (NPAD - N))])
    src_p = src_f.reshape(NCHUNK, 128)
    dst_p = dst_f.reshape(NCHUNK, 128)

    x_pad = jnp.pad(x, ((0, NPAD - N), (0, 0)))
    b1r = b1.reshape(1, D)
    b2r = b2.reshape(1, D)
    fc_b8 = jnp.broadcast_to(fc_b.reshape(1, 1), (8, 128))

    dega, degb = _sc_degree(dst_p)
    xw1 = _tc_matmul(x_pad, W1)
    z1lo, z1hi, dinv = _tc_scale(xw1, dega, degb)
    acc1lo, acc1hi = _sc_agg(z1lo, z1hi, src_p, dst_p)
    xw2, z2lo, z2hi = _tc_layer2(acc1lo, acc1hi, xw1, dinv, b1r, W2)
    acc2lo, acc2hi = _sc_agg(z2lo, z2hi, src_p, dst_p)
    out8 = _tc_final(acc2lo, acc2hi, xw2, dinv, b2r, fc_w, fc_b8)
    return jnp.reshape(out8[0, 0], (1,))
